# Initial kernel scaffold; baseline (speedup 1.0000x reference)
import jax
import jax.numpy as jnp
from jax.experimental import pallas as pl


def kernel(x, edge_index, edge_weights, mW1, mb1, mW2, mb2, sW1, sb1, sW2, sb2):
    raise NotImplementedError("write your pallas kernel here")


# speedup vs baseline: 3.5121x; 3.5121x over previous
import functools

import jax
import jax.numpy as jnp
from jax import lax
from jax.experimental import pallas as pl
from jax.experimental.pallas import tpu as pltpu
from jax.experimental.pallas import tpu_sc as plsc

H = 128
HH = H // 2
N = 10000
NC = 2
NT = 16
SLAB = 624
GRP = 128
NGRP = 8
CHUNK = GRP * NGRP
NCHUNK = 20
EPT = CHUNK * NCHUNK
E_PAD = EPT * NT


def _mlp_body(x_ref, mw1_ref, mb1_ref, mw2_ref, mb2_ref,
              sw1_ref, sb1_ref, sw2_ref, sb2_ref, m2_ref, s2_ref):
    xb = x_ref[...]
    h = jnp.maximum(
        jnp.dot(xb, mw1_ref[...].T, preferred_element_type=jnp.float32)
        + mb1_ref[...], 0.0)
    msg = jnp.dot(h, mw2_ref[...].T, preferred_element_type=jnp.float32) \
        + mb2_ref[...]
    g = jnp.maximum(
        jnp.dot(xb, sw1_ref[...].T, preferred_element_type=jnp.float32)
        + sb1_ref[...], 0.0)
    slf = jnp.dot(g, sw2_ref[...].T, preferred_element_type=jnp.float32) \
        + sb2_ref[...]
    m2_ref[0:N] = msg[:, 0:HH]
    m2_ref[N:2 * N] = msg[:, HH:H]
    s2_ref[0:N] = slf[:, 0:HH]
    s2_ref[N:2 * N] = slf[:, HH:H]


_mlp = pl.pallas_call(
    _mlp_body,
    out_shape=(jax.ShapeDtypeStruct((NC * N, HH), jnp.float32),
               jax.ShapeDtypeStruct((NC * N, HH), jnp.float32)),
)


@functools.partial(
    pl.kernel,
    out_type=jax.ShapeDtypeStruct((NC, N, HH), jnp.float32),
    mesh=plsc.VectorSubcoreMesh(core_axis_name="c", subcore_axis_name="s"),
    compiler_params=pltpu.CompilerParams(needs_layout_passes=False,
                                         use_tc_tiling_on_sc=False),
    scratch_types=[
        pltpu.VMEM((NGRP, GRP), jnp.int32),
        pltpu.VMEM((NGRP, GRP), jnp.int32),
        pltpu.VMEM((CHUNK,), jnp.float32),
        pltpu.VMEM((CHUNK, HH), jnp.float32),
        pltpu.VMEM_SHARED((N, HH), jnp.float32),
        pltpu.SemaphoreType.DMA,
        pltpu.SemaphoreType.DMA,
    ],
)
def _sc_aggr(m2_hbm, s2_hbm, src_hbm, dst_hbm, w_hbm, out_hbm,
             src_v, dst_v, w_v, rows_v, accum, gsem, ssem):
    c = lax.axis_index("c")
    s = lax.axis_index("s")
    pltpu.sync_copy(s2_hbm.at[pl.ds(c * N + s * SLAB, SLAB)],
                    accum.at[pl.ds(s * SLAB, SLAB)])

    @pl.when(s == NT - 1)
    def _init_tail():
        pltpu.sync_copy(s2_hbm.at[pl.ds(c * N + NT * SLAB, N - NT * SLAB)],
                        accum.at[pl.ds(NT * SLAB, N - NT * SLAB)])
    plsc.subcore_barrier()
    row_off = c * N

    def chunk_body(i, carry):
        gbase = s * (EPT // GRP) + i * NGRP
        ebase = s * EPT + i * CHUNK
        pltpu.sync_copy(src_hbm.at[pl.ds(gbase, NGRP)], src_v)
        pltpu.sync_copy(dst_hbm.at[pl.ds(gbase, NGRP)], dst_v)
        pltpu.sync_copy(w_hbm.at[pl.ds(ebase, CHUNK)], w_v)

        def adj(k, _):
            r = k // 8
            l0 = (k % 8) * 16
            src_v[r, pl.ds(l0, 16)] = src_v[r, pl.ds(l0, 16)] + row_off
            return 0
        lax.fori_loop(0, NGRP * 8, adj, 0)

        descs = [
            pltpu.async_copy(m2_hbm.at[src_v.at[j]],
                             rows_v.at[pl.ds(j * GRP, GRP)], gsem)
            for j in range(NGRP)
        ]
        for d in descs:
            d.wait()

        def scale(b4, _):
            for u in range(4):
                b = b4 * 4 + u
                wb = plsc.load_gather(w_v, [jnp.full((16,), b, jnp.int32)])
                for t in range(HH // 16):
                    sl = pl.ds(t * 16, 16)
                    rows_v[b, sl] = rows_v[b, sl] * wb
            return 0
        lax.fori_loop(0, CHUNK // 4, scale, 0)

        descs2 = [
            pltpu.async_copy(rows_v.at[pl.ds(j * GRP, GRP)],
                             accum.at[dst_v.at[j]], ssem, add=True)
            for j in range(NGRP)
        ]
        for d in descs2:
            d.wait()
        return 0

    lax.fori_loop(0, NCHUNK, chunk_body, 0)
    plsc.subcore_barrier()
    pltpu.sync_copy(accum.at[pl.ds(s * SLAB, SLAB)],
                    out_hbm.at[c, pl.ds(s * SLAB, SLAB)])

    @pl.when(s == NT - 1)
    def _out_tail():
        pltpu.sync_copy(accum.at[pl.ds(NT * SLAB, N - NT * SLAB)],
                        out_hbm.at[c, pl.ds(NT * SLAB, N - NT * SLAB)])


def kernel(x, edge_index, edge_weights, mW1, mb1, mW2, mb2, sW1, sb1, sW2, sb2):
    ei = edge_index.astype(jnp.int32)
    e = ei.shape[1]
    pad = E_PAD - e
    src = jnp.concatenate([ei[0], jnp.zeros((pad,), jnp.int32)]).reshape(-1, GRP)
    dst = jnp.concatenate([ei[1], jnp.zeros((pad,), jnp.int32)]).reshape(-1, GRP)
    w = jnp.concatenate([edge_weights.astype(jnp.float32),
                         jnp.zeros((pad,), jnp.float32)])
    m2, s2 = _mlp(x, mW1, mb1.reshape(1, H), mW2, mb2.reshape(1, H),
                  sW1, sb1.reshape(1, H), sW2, sb2.reshape(1, H))
    out2 = _sc_aggr(m2, s2, src, dst, w)
    return out2.transpose(1, 0, 2).reshape(N, H)
